# baseline (speedup 1.0000x reference)
import functools

import jax
import jax.numpy as jnp
import numpy as np
from jax import lax
from jax.experimental import pallas as pl
from jax.experimental.pallas import tpu as pltpu
from jax.experimental.pallas import tpu_sc as plsc

K = 8
EPS = float(np.finfo(np.float32).tiny)
L = 16
N = 4096
NG = N // L
U = 8
R = 512
NC = 2
NS = 16
NW = NC * NS
RPW = R // NW


def _make_sc_slices():
    mesh = plsc.VectorSubcoreMesh(core_axis_name="c", subcore_axis_name="s")

    @functools.partial(
        pl.kernel,
        mesh=mesh,
        compiler_params=pltpu.CompilerParams(needs_layout_passes=False),
        out_type=jax.ShapeDtypeStruct((16, 8, 4, K, N), jnp.float32),
        scratch_types=[
            pltpu.VMEM((N,), jnp.float32),
            pltpu.VMEM((N,), jnp.float32),
            pltpu.VMEM((N,), jnp.float32),
            pltpu.VMEM((K, N), jnp.float32),
            pltpu.SemaphoreType.DMA,
            pltpu.SemaphoreType.DMA,
        ],
    )
    def sc_slices(x_hbm, km_hbm, xva, xvb, tv, bv, sem_in, sem_out):
        c = lax.axis_index("c")
        s = lax.axis_index("s")
        wid = s * NC + c

        zeros = jnp.zeros((L,), jnp.float32)
        ones = jnp.ones((L,), jnp.float32)

        def rbhs(row):
            return row >> 5, (row >> 2) & 7, row & 3

        def fetch(r, xv):
            rb, rh, rs = rbhs(wid * RPW + jnp.minimum(r, RPW - 1))
            pltpu.async_copy(x_hbm.at[rb, rh, rs], xv, sem_in)

        def wait_fetch(xv):
            pltpu.make_async_copy(x_hbm.at[0, 0, 0], xv, sem_in).wait()

        def drain_prev_row():
            pltpu.make_async_copy(bv, km_hbm.at[0, 0, 0],
                                  sem_out).wait()

        def row_body(r, xv, xv_next):
            row = wid * RPW + r
            rb, rh, rs = rbhs(row)
            wait_fetch(xv)
            fetch(r + 1, xv_next)

            @plsc.parallel_loop(0, NG, step=U, carry=(zeros,) * U)
            def max_body(j, accs):
                return tuple(
                    jnp.maximum(accs[k], xv[pl.ds((j + k) * L, L)])
                    for k in range(U)
                )

            mm = max_body
            while len(mm) > 1:
                mm = tuple(jnp.maximum(mm[2 * a], mm[2 * a + 1])
                           for a in range(len(mm) // 2))
            m = jnp.max(mm[0])

            @plsc.parallel_loop(0, NG, step=U, carry=(zeros,) * U)
            def exp_body(j, accs):
                out = []
                for k in range(U):
                    sl = pl.ds((j + k) * L, L)
                    v = jnp.exp(xv[sl] - m)
                    tv[sl] = v
                    out.append(accs[k] + v)
                return tuple(out)

            q = jnp.sum(sum(exp_body[1:], exp_body[0]))

            @pl.when(r != 0)
            def _():
                drain_prev_row()

            r = q
            for i in range(K):
                last = i == K - 1
                beta = ones / r

                @plsc.parallel_loop(0, NG, step=U, carry=(zeros,) * U)
                def iter_body(j, accs, i=i, beta=beta, last=last):
                    out = []
                    for k in range(U):
                        sl = pl.ds((j + k) * L, L)
                        if i == 0:
                            pn = tv[sl] * beta
                        else:
                            p = bv[i - 1, sl]
                            pn = (p - p * p) * beta
                        bv[i, sl] = pn
                        if last:
                            out.append(accs[k])
                        else:
                            out.append(accs[k] + pn * pn)
                    return tuple(out)

                if i == K - 1:
                    pltpu.async_copy(bv, km_hbm.at[rb, rh, rs], sem_out)
                r = 1.0 - jnp.sum(sum(iter_body[1:], iter_body[0]))

        fetch(0, xva)

        def pair_body(rr, carry):
            row_body(2 * rr, xva, xvb)
            row_body(2 * rr + 1, xvb, xva)
            return carry

        lax.fori_loop(0, RPW // 2, pair_body, 0)
        drain_prev_row()
        wait_fetch(xva)

    return sc_slices


_SC_SLICES = _make_sc_slices()


def _tc_khot_body(x_ref, kh_ref):
    x = x_ref[...]
    m = jnp.max(x, axis=-1, keepdims=True)
    t = jnp.exp(x - m)
    kh = jnp.zeros_like(t)
    for i in range(K):
        s = jnp.sum(t, axis=-1, keepdims=True)
        p = t * (1.0 / s)
        kh = kh + p
        if i < K - 1:
            t = t * jnp.maximum(1.0 - p, EPS)
    kh_ref[...] = kh


def _tc_khot(scores):
    return pl.pallas_call(
        _tc_khot_body,
        grid=(16,),
        in_specs=[pl.BlockSpec((1, 8, 4, N), lambda i: (i, 0, 0, 0))],
        out_specs=pl.BlockSpec((1, 8, 4, N), lambda i: (i, 0, 0, 0)),
        out_shape=jax.ShapeDtypeStruct((16, 8, 4, N), jnp.float32),
    )(scores)


def kernel(scores):
    km = _SC_SLICES(scores)
    khot = _tc_khot(scores)
    return khot, km


# speedup vs baseline: 1.0741x; 1.0741x over previous
import functools

import jax
import jax.numpy as jnp
import numpy as np
from jax import lax
from jax.experimental import pallas as pl
from jax.experimental.pallas import tpu as pltpu
from jax.experimental.pallas import tpu_sc as plsc

K = 8
EPS = float(np.finfo(np.float32).tiny)
L = 16
N = 4096
NG = N // L
U = 8
R = 512
NC = 2
NS = 16
NW = NC * NS
RPW = R // NW


def _make_sc_slices():
    mesh = plsc.VectorSubcoreMesh(core_axis_name="c", subcore_axis_name="s")

    @functools.partial(
        pl.kernel,
        mesh=mesh,
        compiler_params=pltpu.CompilerParams(needs_layout_passes=False),
        out_type=jax.ShapeDtypeStruct((16, 8, 4, K, N), jnp.float32),
        scratch_types=[
            pltpu.VMEM((N,), jnp.float32),
            pltpu.VMEM((N,), jnp.float32),
            pltpu.VMEM((N,), jnp.float32),
            pltpu.VMEM((K, N), jnp.float32),
            pltpu.SemaphoreType.DMA,
            pltpu.SemaphoreType.DMA,
        ],
    )
    def sc_slices(x_hbm, km_hbm, xva, xvb, tv, bv, sem_in, sem_out):
        c = lax.axis_index("c")
        s = lax.axis_index("s")
        wid = s * NC + c

        zeros = jnp.zeros((L,), jnp.float32)
        ones = jnp.ones((L,), jnp.float32)

        def rbhs(row):
            return row >> 5, (row >> 2) & 7, row & 3

        def fetch(r, xv):
            rb, rh, rs = rbhs(wid * RPW + jnp.minimum(r, RPW - 1))
            pltpu.async_copy(x_hbm.at[rb, rh, rs], xv, sem_in)

        def wait_fetch(xv):
            pltpu.make_async_copy(x_hbm.at[0, 0, 0], xv, sem_in).wait()

        def drain_prev_row():
            for _ in range(K // 4):
                pltpu.make_async_copy(bv.at[pl.ds(0, 4)],
                                      km_hbm.at[0, 0, 0, pl.ds(0, 4)],
                                      sem_out).wait()

        def row_body(r, xv, xv_next):
            row = wid * RPW + r
            rb, rh, rs = rbhs(row)
            wait_fetch(xv)
            fetch(r + 1, xv_next)

            @plsc.parallel_loop(0, NG, step=U, carry=(zeros,) * U)
            def max_body(j, accs):
                return tuple(
                    jnp.maximum(accs[k], xv[pl.ds((j + k) * L, L)])
                    for k in range(U)
                )

            mm = max_body
            while len(mm) > 1:
                mm = tuple(jnp.maximum(mm[2 * a], mm[2 * a + 1])
                           for a in range(len(mm) // 2))
            m = jnp.max(mm[0])

            @plsc.parallel_loop(0, NG, step=U, carry=(zeros,) * U)
            def exp_body(j, accs):
                out = []
                for k in range(U):
                    sl = pl.ds((j + k) * L, L)
                    v = jnp.exp(xv[sl] - m)
                    tv[sl] = v
                    out.append(accs[k] + v)
                return tuple(out)

            q = jnp.sum(sum(exp_body[1:], exp_body[0]))

            @pl.when(r != 0)
            def _():
                drain_prev_row()

            r = q
            for i in range(K):
                last = i == K - 1
                beta = ones / r

                @plsc.parallel_loop(0, NG, step=U, carry=(zeros,) * U)
                def iter_body(j, accs, i=i, beta=beta, last=last):
                    out = []
                    for k in range(U):
                        sl = pl.ds((j + k) * L, L)
                        if i == 0:
                            pn = tv[sl] * beta
                        else:
                            p = bv[i - 1, sl]
                            pn = (p - p * p) * beta
                        bv[i, sl] = pn
                        if last:
                            out.append(accs[k])
                        else:
                            out.append(accs[k] + pn * pn)
                    return tuple(out)

                if i % 4 == 3:
                    pltpu.async_copy(bv.at[pl.ds(i - 3, 4)],
                                     km_hbm.at[rb, rh, rs, pl.ds(i - 3, 4)],
                                     sem_out)
                r = 1.0 - jnp.sum(sum(iter_body[1:], iter_body[0]))

        fetch(0, xva)

        def pair_body(rr, carry):
            row_body(2 * rr, xva, xvb)
            row_body(2 * rr + 1, xvb, xva)
            return carry

        lax.fori_loop(0, RPW // 2, pair_body, 0)
        drain_prev_row()
        wait_fetch(xva)

    return sc_slices


_SC_SLICES = _make_sc_slices()


def _tc_khot_body(x_ref, kh_ref):
    x = x_ref[...]
    m = jnp.max(x, axis=-1, keepdims=True)
    t = jnp.exp(x - m)
    kh = jnp.zeros_like(t)
    for i in range(K):
        s = jnp.sum(t, axis=-1, keepdims=True)
        p = t * (1.0 / s)
        kh = kh + p
        if i < K - 1:
            t = t * jnp.maximum(1.0 - p, EPS)
    kh_ref[...] = kh


def _tc_khot(scores):
    return pl.pallas_call(
        _tc_khot_body,
        grid=(16,),
        in_specs=[pl.BlockSpec((1, 8, 4, N), lambda i: (i, 0, 0, 0))],
        out_specs=pl.BlockSpec((1, 8, 4, N), lambda i: (i, 0, 0, 0)),
        out_shape=jax.ShapeDtypeStruct((16, 8, 4, N), jnp.float32),
    )(scores)


def kernel(scores):
    km = _SC_SLICES(scores)
    khot = _tc_khot(scores)
    return khot, km
